# Initial kernel scaffold; baseline (speedup 1.0000x reference)
import jax
import jax.numpy as jnp
from jax.experimental import pallas as pl


def kernel(logp, scores, end_flag, hyps):
    raise NotImplementedError("write your pallas kernel here")


# speedup vs baseline: 1.4511x; 1.4511x over previous
import functools

import jax
import jax.numpy as jnp
from jax import lax
from jax.experimental import pallas as pl
from jax.experimental.pallas import tpu as pltpu
from jax.experimental.pallas import tpu_sc as plsc

_BMS = 10
_EOS = 2
_NEG_INF = -1e30
_MINF = float("-inf")

_RNS = 640
_VOCAB = 100000
_BS = 64
_NW = 32
_ROWS_PER_W = _RNS // _NW
_GRP = 10
_NGRP = _VOCAB // (16 * _GRP)


def _sc_topk_build():
    mesh = plsc.VectorSubcoreMesh(core_axis_name="c", subcore_axis_name="s")

    @functools.partial(
        pl.kernel,
        mesh=mesh,
        out_type=[
            jax.ShapeDtypeStruct((_RNS, 16), jnp.float32),
            jax.ShapeDtypeStruct((_RNS, 16), jnp.int32),
        ],
        scratch_types=[
            pltpu.VMEM((_VOCAB,), jnp.float32),
            pltpu.VMEM((16,), jnp.float32),
            pltpu.VMEM((16,), jnp.int32),
            pltpu.VMEM((16,), jnp.float32),
        ],
    )
    def sc_topk(logp_hbm, vals_hbm, idxs_hbm, row_buf, bv_ref, bi_ref, tv_ref):
        wid = lax.axis_index("s") * 2 + lax.axis_index("c")
        lane = lax.iota(jnp.int32, 16)
        nine = jnp.full((16,), _BMS - 1, jnp.int32)

        def gat(x, perm):
            return x.at[perm].get(mode="promise_in_bounds")

        def bmax(x):
            for d in (8, 4, 2, 1):
                x = jnp.maximum(x, gat(x, lane ^ d))
            return x

        def cmp_exchange(x, xi, j, take_max_i):
            perm = lane ^ j
            px = gat(x, perm)
            pi = gat(xi, perm)
            gt = jnp.where(x > px, 1, 0)
            eq = jnp.where(x == px, 1, 0)
            lt = jnp.where(xi < pi, 1, 0)
            pref = gt | (eq & lt)
            keep = (1 - (take_max_i ^ pref)) == 1
            return jnp.where(keep, x, px), jnp.where(keep, xi, pi)

        def sort16_desc(x, xi):
            for kshift in (1, 2, 3, 4):
                k = 1 << kshift
                j = k // 2
                while j >= 1:
                    bk = (lane >> kshift) & 1
                    bj = jnp.where((lane & j) == 0, 1, 0)
                    x, xi = cmp_exchange(x, xi, j, bj ^ bk)
                    j //= 2
            return x, xi

        def clean_desc(x, xi):
            for j in (8, 4, 2, 1):
                bj = jnp.where((lane & j) == 0, 1, 0)
                x, xi = cmp_exchange(x, xi, j, bj)
            return x, xi

        def row_body(i, _):
            r = wid * _ROWS_PER_W + i
            pltpu.sync_copy(logp_hbm.at[r], row_buf)

            bv_ref[...] = jnp.full((16,), _MINF, jnp.float32)
            bi_ref[...] = jnp.zeros((16,), jnp.int32)
            tv_ref[...] = jnp.full((16,), _MINF, jnp.float32)

            def group_body(g, _c):
                base = g * (16 * _GRP)
                vs = [row_buf[pl.ds(base + 16 * j, 16)] for j in range(_GRP)]
                gmax = vs[0]
                for j in range(1, _GRP):
                    gmax = jnp.maximum(gmax, vs[j])
                am = bmax(gmax)
                tvv = tv_ref[...]

                @pl.when(am[0] > tvv[0])
                def _():
                    for j in range(_GRP):
                        v = vs[j]
                        vmx = bmax(v)
                        tnow = tv_ref[...]

                        @pl.when(vmx[0] > tnow[0])
                        def _():
                            tv_in = tv_ref[...]
                            vm = jnp.where(v > tv_in, v, _MINF)
                            vidx = base + 16 * j + lane
                            sv, si = sort16_desc(vm, vidx)
                            bv = bv_ref[...]
                            bi = bi_ref[...]
                            rbv = lax.rev(bv, (0,))
                            rbi = lax.rev(bi, (0,))
                            take = sv > rbv
                            mv = jnp.where(take, sv, rbv)
                            mi = jnp.where(take, si, rbi)
                            nbv, nbi = clean_desc(mv, mi)
                            bv_ref[...] = nbv
                            bi_ref[...] = nbi
                            tv_ref[...] = gat(nbv, nine)

                return 0

            lax.fori_loop(0, _NGRP, group_body, 0)
            pltpu.sync_copy(bv_ref, vals_hbm.at[r])
            pltpu.sync_copy(bi_ref, idxs_hbm.at[r])
            return 0

        lax.fori_loop(0, _ROWS_PER_W, row_body, 0)

    return sc_topk


_sc_topk_cache = []


def _sc_topk(logp):
    if not _sc_topk_cache:
        _sc_topk_cache.append(_sc_topk_build())
    return _sc_topk_cache[0](logp)


def _stage2_body(tv_ref, ti_ref, sc_ref, ef_ref, ns_ref, pred_ref, bh_ref):
    tv = tv_ref[...]
    ti = ti_ref[...]
    sc = sc_ref[...]
    fin = ef_ref[...] != 0
    li = lax.broadcasted_iota(jnp.int32, (_BS, 160), 1)
    col = li % 16
    row_b = lax.broadcasted_iota(jnp.int32, (_BS, 16), 0)

    tl = jnp.where(fin & (col >= 1), _NEG_INF, tv)
    tl = jnp.where(fin & (col == 0), 0.0, tl)
    tl = jnp.where(col >= _BMS, _NEG_INF, tl)
    cand = sc + tl
    pid = jnp.where(fin, _EOS, ti).astype(jnp.float32)

    for k in range(_BMS):
        m = jnp.max(cand, axis=1, keepdims=True)
        jsel = jnp.min(jnp.where(cand == m, li, 10_000), axis=1, keepdims=True)
        first = li == jsel
        p = jnp.sum(jnp.where(first, pid, 0.0), axis=1,
                    keepdims=True)
        ns_ref[:, k:k + 1] = m
        pred_ref[:, k:k + 1] = p
        bh_ref[:, k:k + 1] = row_b[:, :1] * _BMS + jsel // 16
        cand = jnp.where(first, _MINF, cand)


_stage2 = pl.pallas_call(
    _stage2_body,
    out_shape=[
        jax.ShapeDtypeStruct((_BS, 16), jnp.float32),
        jax.ShapeDtypeStruct((_BS, 16), jnp.float32),
        jax.ShapeDtypeStruct((_BS, 16), jnp.int32),
    ],
)


def _gather_body(bh_ref, hyps_ref, pred_ref, hyp_out_ref, end_ref):
    bh = bh_ref[...]
    src = lax.broadcasted_iota(jnp.int32, (_RNS, _RNS), 1)
    g = (bh == src).astype(jnp.float32)
    hyp_out_ref[...] = jnp.dot(g, hyps_ref[...],
                               preferred_element_type=jnp.float32)
    end_ref[...] = (pred_ref[...] == _EOS).astype(jnp.int32)


def _gather_call(cur_len):
    return pl.pallas_call(
        _gather_body,
        out_shape=[
            jax.ShapeDtypeStruct((_RNS, cur_len), jnp.float32),
            jax.ShapeDtypeStruct((_RNS, 1), jnp.int32),
        ],
    )


def kernel(logp, scores, end_flag, hyps):
    cur_len = hyps.shape[1]
    vals16, idxs16 = _sc_topk(logp)

    tv2 = vals16.reshape(_BS, 160)
    ti2 = idxs16.reshape(_BS, 160)
    sc2 = jnp.repeat(scores.reshape(_BS, _BMS), 16, axis=1)
    ef2 = jnp.repeat(end_flag.reshape(_BS, _BMS).astype(jnp.int32), 16, axis=1)

    ns64, pred64, bh64 = _stage2(tv2, ti2, sc2, ef2)

    new_scores = ns64[:, :_BMS].reshape(_RNS, 1)
    pred = pred64[:, :_BMS].reshape(_RNS, 1).astype(hyps.dtype)
    bh = bh64[:, :_BMS].reshape(_RNS, 1)

    hyp_g_f, end_i = _gather_call(cur_len)(bh, hyps.astype(jnp.float32),
                                           pred.astype(jnp.int32))
    new_hyps = jnp.concatenate([hyp_g_f.astype(hyps.dtype), pred], axis=1)
    new_end_flag = end_i.astype(bool)
    return new_scores, new_hyps, new_end_flag


# speedup vs baseline: 1.4772x; 1.0180x over previous
import functools

import jax
import jax.numpy as jnp
from jax import lax
from jax.experimental import pallas as pl
from jax.experimental.pallas import tpu as pltpu
from jax.experimental.pallas import tpu_sc as plsc

_BMS = 10
_EOS = 2
_NEG_INF = -1e30
_MINF = float("-inf")

_RNS = 640
_VOCAB = 100000
_BS = 64
_NW = 32
_ROWS_PER_W = _RNS // _NW
_GRP = 10
_NGRP = _VOCAB // (16 * _GRP)
_L1 = 16
_NL1 = 39
_VPAD = _VOCAB


def _sc_topk_build():
    mesh = plsc.VectorSubcoreMesh(core_axis_name="c", subcore_axis_name="s")

    @functools.partial(
        pl.kernel,
        mesh=mesh,
        out_type=[
            jax.ShapeDtypeStruct((_RNS, 16), jnp.float32),
            jax.ShapeDtypeStruct((_RNS, 16), jnp.int32),
        ],
        scratch_types=[
            pltpu.VMEM((_VOCAB,), jnp.float32),
            pltpu.VMEM((_NL1 * _L1 * 16,), jnp.float32),
            pltpu.VMEM((_NL1 * 16,), jnp.float32),
            pltpu.VMEM((16,), jnp.float32),
            pltpu.VMEM((16,), jnp.int32),
            pltpu.VMEM((16,), jnp.float32),
        ],
    )
    def sc_topk(logp_hbm, vals_hbm, idxs_hbm, row_buf, gmax_ref, l1_ref,
                bv_ref, bi_ref, tv_ref):
        wid = lax.axis_index("s") * 2 + lax.axis_index("c")
        lane = lax.iota(jnp.int32, 16)
        nine = jnp.full((16,), _BMS - 1, jnp.int32)

        def gat(x, perm):
            return x.at[perm].get(mode="promise_in_bounds")

        def bmax(x):
            for d in (8, 4, 2, 1):
                x = jnp.maximum(x, gat(x, lane ^ d))
            return x

        def cmp_exchange(x, xi, j, take_max_i):
            perm = lane ^ j
            px = gat(x, perm)
            pi = gat(xi, perm)
            gt = jnp.where(x > px, 1, 0)
            eq = jnp.where(x == px, 1, 0)
            lt = jnp.where(xi < pi, 1, 0)
            pref = gt | (eq & lt)
            keep = (1 - (take_max_i ^ pref)) == 1
            return jnp.where(keep, x, px), jnp.where(keep, xi, pi)

        def sort16_desc(x, xi):
            for kshift in (1, 2, 3, 4):
                k = 1 << kshift
                j = k // 2
                while j >= 1:
                    bk = (lane >> kshift) & 1
                    bj = jnp.where((lane & j) == 0, 1, 0)
                    x, xi = cmp_exchange(x, xi, j, bj ^ bk)
                    j //= 2
            return x, xi

        def clean_desc(x, xi):
            for j in (8, 4, 2, 1):
                bj = jnp.where((lane & j) == 0, 1, 0)
                x, xi = cmp_exchange(x, xi, j, bj)
            return x, xi

        def merge_vec(v, vbase):
            tv_in = tv_ref[...]
            vm = jnp.where(v > tv_in, v, _MINF)
            vidx = vbase + lane
            sv, si = sort16_desc(vm, vidx)
            bv = bv_ref[...]
            bi = bi_ref[...]
            rbv = lax.rev(bv, (0,))
            rbi = lax.rev(bi, (0,))
            take = sv > rbv
            mv = jnp.where(take, sv, rbv)
            mi = jnp.where(take, si, rbi)
            nbv, nbi = clean_desc(mv, mi)
            bv_ref[...] = nbv
            bi_ref[...] = nbi
            tv_ref[...] = gat(nbv, nine)

        def row_body(i, _):
            r = wid * _ROWS_PER_W + i
            pltpu.sync_copy(logp_hbm.at[r], row_buf)

            bv_ref[...] = jnp.full((16,), _MINF, jnp.float32)
            bi_ref[...] = jnp.zeros((16,), jnp.int32)
            tv_ref[...] = jnp.full((16,), _MINF, jnp.float32)

            def build_body(g, _c):
                l1 = None
                for sg in range(_L1):
                    base = (g * _L1 + sg) * (16 * _GRP)
                    gm = row_buf[pl.ds(base, 16)]
                    for j in range(1, _GRP):
                        gm = jnp.maximum(gm, row_buf[pl.ds(base + 16 * j, 16)])
                    gmax_ref[pl.ds((g * _L1 + sg) * 16, 16)] = gm
                    l1 = gm if l1 is None else jnp.maximum(l1, gm)
                l1_ref[pl.ds(g * 16, 16)] = l1
                return 0

            lax.fori_loop(0, _NL1, build_body, 0)

            def scan_body(g, _c):
                l1 = l1_ref[pl.ds(g * 16, 16)]
                am = bmax(l1)
                tvv = tv_ref[...]

                @pl.when(am[0] > tvv[0])
                def _():
                    def sub_body(sg, _c2):
                        gidx = g * _L1 + sg
                        gm = gmax_ref[pl.ds(gidx * 16, 16)]
                        gb = bmax(gm)
                        t2 = tv_ref[...]

                        @pl.when(gb[0] > t2[0])
                        def _():
                            def vec_body(j, _c3):
                                vbase = gidx * (16 * _GRP) + 16 * j
                                v = row_buf[pl.ds(vbase, 16)]
                                vb = bmax(v)
                                t3 = tv_ref[...]

                                @pl.when(vb[0] > t3[0])
                                def _():
                                    merge_vec(v, vbase)

                                return 0

                            lax.fori_loop(0, _GRP, vec_body, 0)

                        return 0

                    lax.fori_loop(0, _L1, sub_body, 0)

                return 0

            lax.fori_loop(0, _NL1, scan_body, 0)

            lbase = _NL1 * _L1 * (16 * _GRP)
            lgm = row_buf[pl.ds(lbase, 16)]
            for j in range(1, _GRP):
                lgm = jnp.maximum(lgm, row_buf[pl.ds(lbase + 16 * j, 16)])
            lb = bmax(lgm)
            lt = tv_ref[...]

            @pl.when(lb[0] > lt[0])
            def _():
                def lvec_body(j, _c3):
                    vbase = lbase + 16 * j
                    v = row_buf[pl.ds(vbase, 16)]
                    vb = bmax(v)
                    t3 = tv_ref[...]

                    @pl.when(vb[0] > t3[0])
                    def _():
                        merge_vec(v, vbase)

                    return 0

                lax.fori_loop(0, _GRP, lvec_body, 0)

            pltpu.sync_copy(bv_ref, vals_hbm.at[r])
            pltpu.sync_copy(bi_ref, idxs_hbm.at[r])
            return 0

        lax.fori_loop(0, _ROWS_PER_W, row_body, 0)

    return sc_topk


_sc_topk_cache = []


def _sc_topk(logp):
    if not _sc_topk_cache:
        _sc_topk_cache.append(_sc_topk_build())
    return _sc_topk_cache[0](logp)


def _stage2_body(tv_ref, ti_ref, sc_ref, ef_ref, ns_ref, pred_ref, bh_ref):
    tv = tv_ref[...]
    ti = ti_ref[...]
    sc = sc_ref[...]
    fin = ef_ref[...] != 0
    li = lax.broadcasted_iota(jnp.int32, (_BS, 160), 1)
    col = li % 16
    row_b = lax.broadcasted_iota(jnp.int32, (_BS, 16), 0)

    tl = jnp.where(fin & (col >= 1), _NEG_INF, tv)
    tl = jnp.where(fin & (col == 0), 0.0, tl)
    tl = jnp.where(col >= _BMS, _NEG_INF, tl)
    cand = sc + tl
    pid = jnp.where(fin, _EOS, ti).astype(jnp.float32)

    for k in range(_BMS):
        m = jnp.max(cand, axis=1, keepdims=True)
        jsel = jnp.min(jnp.where(cand == m, li, 10_000), axis=1, keepdims=True)
        first = li == jsel
        p = jnp.sum(jnp.where(first, pid, 0.0), axis=1,
                    keepdims=True)
        ns_ref[:, k:k + 1] = m
        pred_ref[:, k:k + 1] = p
        bh_ref[:, k:k + 1] = row_b[:, :1] * _BMS + jsel // 16
        cand = jnp.where(first, _MINF, cand)


_stage2 = pl.pallas_call(
    _stage2_body,
    out_shape=[
        jax.ShapeDtypeStruct((_BS, 16), jnp.float32),
        jax.ShapeDtypeStruct((_BS, 16), jnp.float32),
        jax.ShapeDtypeStruct((_BS, 16), jnp.int32),
    ],
)


def _gather_body(bh_ref, hyps_ref, pred_ref, hyp_out_ref, end_ref):
    bh = bh_ref[...]
    src = lax.broadcasted_iota(jnp.int32, (_RNS, _RNS), 1)
    g = (bh == src).astype(jnp.float32)
    hyp_out_ref[...] = jnp.dot(g, hyps_ref[...],
                               preferred_element_type=jnp.float32)
    end_ref[...] = (pred_ref[...] == _EOS).astype(jnp.int32)


def _gather_call(cur_len):
    return pl.pallas_call(
        _gather_body,
        out_shape=[
            jax.ShapeDtypeStruct((_RNS, cur_len), jnp.float32),
            jax.ShapeDtypeStruct((_RNS, 1), jnp.int32),
        ],
    )


def kernel(logp, scores, end_flag, hyps):
    cur_len = hyps.shape[1]
    vals16, idxs16 = _sc_topk(logp)

    tv2 = vals16.reshape(_BS, 160)
    ti2 = idxs16.reshape(_BS, 160)
    sc2 = jnp.repeat(scores.reshape(_BS, _BMS), 16, axis=1)
    ef2 = jnp.repeat(end_flag.reshape(_BS, _BMS).astype(jnp.int32), 16, axis=1)

    ns64, pred64, bh64 = _stage2(tv2, ti2, sc2, ef2)

    new_scores = ns64[:, :_BMS].reshape(_RNS, 1)
    pred = pred64[:, :_BMS].reshape(_RNS, 1).astype(hyps.dtype)
    bh = bh64[:, :_BMS].reshape(_RNS, 1)

    hyp_g_f, end_i = _gather_call(cur_len)(bh, hyps.astype(jnp.float32),
                                           pred.astype(jnp.int32))
    new_hyps = jnp.concatenate([hyp_g_f.astype(hyps.dtype), pred], axis=1)
    new_end_flag = end_i.astype(bool)
    return new_scores, new_hyps, new_end_flag
